# Initial kernel scaffold; baseline (speedup 1.0000x reference)
import jax
import jax.numpy as jnp
from jax.experimental import pallas as pl


def kernel(cls_scores, bbox_preds, objectness, priors):
    raise NotImplementedError("write your pallas kernel here")


# speedup vs baseline: 2.1487x; 2.1487x over previous
import functools

import jax
import jax.numpy as jnp
from jax.experimental import pallas as pl
from jax.experimental.pallas import tpu as pltpu

_N = 8400
_NP = 8448
_R = 66
_C = 80
_K = 100
_SCORE_THR = 0.05
_IOU_THR = 0.65


def _nms_body(cls_ref, bp_ref, pri_ref, obj_ref, dets_ref, lbl_ref, keep_ref,
              sup_ref):
    f32 = jnp.float32
    i32 = jnp.int32
    riota = jax.lax.broadcasted_iota(i32, (_R, 128), 0)
    liota = jax.lax.broadcasted_iota(i32, (_R, 128), 1)
    giota = riota * 128 + liota

    def cls_step(c, carry):
        m, lbl = carry
        x = cls_ref[c]
        gt = x > m
        return jnp.where(gt, x, m), jnp.where(gt, c, lbl)

    m0 = jnp.full((_R, 128), -1e30, f32)
    l0 = jnp.zeros((_R, 128), i32)
    m, lbl = jax.lax.fori_loop(0, _C, cls_step, (m0, l0))

    score = jax.nn.sigmoid(m) * jax.nn.sigmoid(obj_ref[:])
    masked = jnp.where(score >= _SCORE_THR, score, -1.0)
    masked = jnp.where(giota >= _N, -2.0, masked)

    st = pri_ref[2]
    cx = bp_ref[0] * st + pri_ref[0]
    cy = bp_ref[1] * st + pri_ref[1]
    w2 = jnp.exp(bp_ref[2]) * st * 0.5
    h2 = jnp.exp(bp_ref[3]) * st * 0.5
    bx1 = cx - w2
    by1 = cy - h2
    bx2 = cx + w2
    by2 = cy + h2

    kiota = jax.lax.broadcasted_iota(i32, (1, 128), 1)
    lblf = lbl.astype(f32)

    def sel(k, s):
        cur, x1r, y1r, x2r, y2r, scr, lbr = s
        mx = jnp.max(cur)
        gidx = jnp.min(jnp.where(cur == mx, giota, jnp.int32(1 << 30)))
        eq = giota == gidx
        cur = jnp.where(eq, -3.0, cur)
        kx = kiota == k

        def pick(ch):
            return jnp.sum(jnp.where(eq, ch, 0.0))

        x1r = jnp.where(kx, pick(bx1), x1r)
        y1r = jnp.where(kx, pick(by1), y1r)
        x2r = jnp.where(kx, pick(bx2), x2r)
        y2r = jnp.where(kx, pick(by2), y2r)
        lbr = jnp.where(kx, pick(lblf), lbr)
        scr = jnp.where(kx, mx, scr)
        return cur, x1r, y1r, x2r, y2r, scr, lbr

    z = jnp.zeros((1, 128), f32)
    _, x1r, y1r, x2r, y2r, scr, lbr = jax.lax.fori_loop(
        0, _K, sel, (masked, z, z, z, z, z, z))

    sl = jax.lax.broadcasted_iota(i32, (8, 128), 0)
    M = jnp.where(sl == 0, x1r, 0.0)
    M = jnp.where(sl == 1, y1r, M)
    M = jnp.where(sl == 2, x2r, M)
    M = jnp.where(sl == 3, y2r, M)
    M = jnp.where(sl == 4, lbr, M)
    T = jnp.transpose(M)
    x1c = T[:, 0:1]
    y1c = T[:, 1:2]
    x2c = T[:, 2:3]
    y2c = T[:, 3:4]
    lbc = T[:, 4:5]

    ix1 = jnp.maximum(x1c, x1r)
    iy1 = jnp.maximum(y1c, y1r)
    ix2 = jnp.minimum(x2c, x2r)
    iy2 = jnp.minimum(y2c, y2r)
    inter = jnp.maximum(ix2 - ix1, 0.0) * jnp.maximum(iy2 - iy1, 0.0)
    ar = (x2r - x1r) * (y2r - y1r)
    ac = (x2c - x1c) * (y2c - y1c)
    iou = inter / (ac + ar - inter + 1e-8)
    sup = ((iou >= _IOU_THR) & (lbc == lbr)).astype(f32)
    sup_ref[:, :] = sup

    keep0 = (scr > 0.0).astype(f32)
    lanef = kiota

    def nms(i, keep):
        row = sup_ref[pl.ds(i, 1), :]
        ki = jnp.sum(jnp.where(lanef == i, keep, 0.0))
        rm = row * ki * (lanef > i).astype(f32)
        return keep * (1.0 - rm)

    keep = jax.lax.fori_loop(0, _K, nms, keep0)

    M2 = jnp.where(sl == 0, x1r, 0.0)
    M2 = jnp.where(sl == 1, y1r, M2)
    M2 = jnp.where(sl == 2, x2r, M2)
    M2 = jnp.where(sl == 3, y2r, M2)
    M2 = jnp.where(sl == 4, scr, M2)
    M2 = jnp.where(sl == 5, lbr, M2)
    M2 = jnp.where(sl == 6, keep, M2)
    T2 = jnp.transpose(M2)
    dets_ref[:, :] = T2[0:_K, 0:5]
    lbl_ref[:, :] = T2[0:_K, 5:6].astype(i32)
    keep_ref[:, :] = (T2[0:_K, 6:7] > 0.0).astype(i32)


@jax.jit
def kernel(cls_scores, bbox_preds, objectness, priors):
    f32 = jnp.float32
    clsT = jnp.pad(cls_scores[0].T, ((0, 0), (0, _NP - _N)),
                   constant_values=-1e30).reshape(_C, _R, 128)
    bpT = jnp.pad(bbox_preds[0].T, ((0, 0), (0, _NP - _N))).reshape(4, _R, 128)
    priT = jnp.pad(priors.T, ((0, 0), (0, _NP - _N)),
                   constant_values=1.0).reshape(4, _R, 128)
    objp = jnp.pad(objectness[0], (0, _NP - _N),
                   constant_values=-100.0).reshape(_R, 128)

    dets, lbl2, keep2 = pl.pallas_call(
        _nms_body,
        out_shape=[
            jax.ShapeDtypeStruct((_K, 5), f32),
            jax.ShapeDtypeStruct((_K, 1), jnp.int32),
            jax.ShapeDtypeStruct((_K, 1), jnp.int32),
        ],
        scratch_shapes=[pltpu.VMEM((128, 128), f32)],
    )(clsT, bpT, priT, objp)
    return dets, lbl2.reshape(_K), keep2.reshape(_K) != 0


# speedup vs baseline: 2.6356x; 1.2266x over previous
import functools

import jax
import jax.numpy as jnp
from jax.experimental import pallas as pl
from jax.experimental.pallas import tpu as pltpu

_N = 8400
_NP = 8448
_R = 66
_RP = 72
_C = 80
_K = 100
_SCORE_THR = 0.05
_IOU_THR = 0.65
_BIGI = 1 << 20


def _tpose(x):
    n = x.shape[1]
    eye = (jax.lax.broadcasted_iota(jnp.int32, (n, n), 0)
           == jax.lax.broadcasted_iota(jnp.int32, (n, n), 1)).astype(jnp.float32)
    return jax.lax.dot_general(eye, x, (((1,), (1,)), ((), ())),
                               precision=jax.lax.Precision.HIGHEST,
                               preferred_element_type=jnp.float32)


def _nms_body(cls_ref, bp_ref, pri_ref, obj_ref, dets_ref, lbl_ref, keep_ref,
              sup_ref):
    f32 = jnp.float32
    i32 = jnp.int32
    riota = jax.lax.broadcasted_iota(i32, (_R, 128), 0)
    liota = jax.lax.broadcasted_iota(i32, (_R, 128), 1)
    giota = riota * 128 + liota
    kiota = jax.lax.broadcasted_iota(i32, (1, 128), 1)

    m = cls_ref[0]
    lblf = jnp.zeros((_R, 128), f32)
    for c in range(1, _C):
        x = cls_ref[c]
        gt = x > m
        m = jnp.where(gt, x, m)
        lblf = jnp.where(gt, f32(c), lblf)

    score = jax.nn.sigmoid(m) * jax.nn.sigmoid(obj_ref[:])
    masked = jnp.where(score >= _SCORE_THR, score, -1.0)
    masked = jnp.where(giota >= _N, -2.0, masked)

    st = pri_ref[2]
    cx = bp_ref[0] * st + pri_ref[0]
    cy = bp_ref[1] * st + pri_ref[1]
    w2 = jnp.exp(bp_ref[2]) * st * 0.5
    h2 = jnp.exp(bp_ref[3]) * st * 0.5
    bx1 = cx - w2
    by1 = cy - h2
    bx2 = cx + w2
    by2 = cy + h2

    def sel(k, s):
        cur, idxr, scr = s
        mx = jnp.max(cur)
        rowc = jnp.min(jnp.where(cur == mx, riota, _BIGI), axis=0,
                       keepdims=True)
        widx = jnp.min(rowc * 128 + kiota)
        kx = kiota == k
        idxr = jnp.where(kx, widx, idxr)
        scr = jnp.where(kx, mx, scr)
        cur = jnp.where(giota == widx, -3.0, cur)
        return cur, idxr, scr

    zf = jnp.zeros((1, 128), f32)
    zi = jnp.zeros((1, 128), i32)
    _, idxr, scr = jax.lax.fori_loop(0, _K, sel, (masked, zi, zf))

    sl8 = jax.lax.broadcasted_iota(i32, (8, 128), 0)
    M = jnp.where(sl8 == 0, idxr.astype(f32), 0.0)
    M = jnp.where(sl8 == 1, scr, M)
    T = _tpose(M)
    idxc = T[:, 0:1].astype(i32)
    scol = T[:, 1:2]
    rvec = idxc >> 7
    lvec = idxc & 127
    ohr = (jax.lax.broadcasted_iota(i32, (128, _RP), 1) == rvec).astype(f32)
    ohl = (jax.lax.broadcasted_iota(i32, (128, 128), 1) == lvec).astype(f32)
    padrows = jnp.zeros((_RP - _R, 128), f32)
    ones1 = jnp.ones((128, 1), f32)

    def gather_col(ch):
        chp = jnp.concatenate([ch, padrows], axis=0)
        g = jax.lax.dot_general(ohr, chp, (((1,), (0,)), ((), ())),
                                precision=jax.lax.Precision.HIGHEST,
                                preferred_element_type=f32)
        return jax.lax.dot_general(g * ohl, ones1, (((1,), (0,)), ((), ())),
                                   precision=jax.lax.Precision.HIGHEST,
                                   preferred_element_type=f32)

    x1c = gather_col(bx1)
    y1c = gather_col(by1)
    x2c = gather_col(bx2)
    y2c = gather_col(by2)
    lbc = gather_col(lblf)

    C5 = jnp.concatenate([x1c, y1c, x2c, y2c, lbc,
                          jnp.zeros((128, 3), f32)], axis=1)
    T2 = _tpose(C5)
    x1r = T2[0:1, :]
    y1r = T2[1:2, :]
    x2r = T2[2:3, :]
    y2r = T2[3:4, :]
    lbr = T2[4:5, :]

    ix1 = jnp.maximum(x1c, x1r)
    iy1 = jnp.maximum(y1c, y1r)
    ix2 = jnp.minimum(x2c, x2r)
    iy2 = jnp.minimum(y2c, y2r)
    inter = jnp.maximum(ix2 - ix1, 0.0) * jnp.maximum(iy2 - iy1, 0.0)
    ar = (x2r - x1r) * (y2r - y1r)
    ac = (x2c - x1c) * (y2c - y1c)
    iou = inter / (ac + ar - inter + 1e-8)
    sup = ((iou >= _IOU_THR) & (lbc == lbr)).astype(f32)
    sup_ref[:, :] = sup

    keep0 = (scr > 0.0).astype(f32)

    def nms(i, keep):
        row = sup_ref[pl.ds(i, 1), :]
        ki = jnp.sum(jnp.where(kiota == i, keep, 0.0))
        rm = row * ki * (kiota > i).astype(f32)
        return keep * (1.0 - rm)

    keep = jax.lax.fori_loop(0, _K, nms, keep0)

    dets_ref[:, 0:1] = x1c[0:_K]
    dets_ref[:, 1:2] = y1c[0:_K]
    dets_ref[:, 2:3] = x2c[0:_K]
    dets_ref[:, 3:4] = y2c[0:_K]
    dets_ref[:, 4:5] = scol[0:_K]
    lbl_ref[:, :] = lbr
    keep_ref[:, :] = keep


@jax.jit
def kernel(cls_scores, bbox_preds, objectness, priors):
    f32 = jnp.float32
    clsT = jnp.pad(cls_scores[0].T, ((0, 0), (0, _NP - _N)),
                   constant_values=-1e30).reshape(_C, _R, 128)
    bpT = jnp.pad(bbox_preds[0].T, ((0, 0), (0, _NP - _N))).reshape(4, _R, 128)
    priT = jnp.pad(priors.T, ((0, 0), (0, _NP - _N)),
                   constant_values=1.0).reshape(4, _R, 128)
    objp = jnp.pad(objectness[0], (0, _NP - _N),
                   constant_values=-100.0).reshape(_R, 128)

    dets, lblrow, keeprow = pl.pallas_call(
        _nms_body,
        out_shape=[
            jax.ShapeDtypeStruct((_K, 5), f32),
            jax.ShapeDtypeStruct((1, 128), f32),
            jax.ShapeDtypeStruct((1, 128), f32),
        ],
        scratch_shapes=[pltpu.VMEM((128, 128), f32)],
    )(clsT, bpT, priT, objp)
    return (dets, lblrow[0, :_K].astype(jnp.int32), keeprow[0, :_K] != 0.0)
